# baseline (speedup 1.0000x reference)
import functools

import jax
import jax.numpy as jnp
from jax import lax
from jax.experimental import pallas as pl
from jax.experimental.pallas import tpu as pltpu
from jax.experimental.pallas import tpu_sc as plsc

D_MODEL = 1024
MAX_OBS = 2000
HALF = D_MODEL // 2


def _prep_body(mask_ref, w1_ref, b1_ref, w2_ref, b2_ref,
               obs_ref, gap_ref, table_ref):
    B, T = mask_ref.shape
    valid = 1 - mask_ref[...]
    pos = lax.broadcasted_iota(jnp.int32, (B, T), 1)

    c = valid
    k = 1
    while k < T:
        shifted = jnp.concatenate(
            [jnp.zeros((B, k), jnp.int32), c[:, : T - k]], axis=1)
        c = c + shifted
        k *= 2
    obs_ref[...] = jnp.clip(c - 1, 0, MAX_OBS - 1)

    m = jnp.where(valid == 1, pos, -1)
    k = 1
    while k < T:
        shifted = jnp.concatenate(
            [jnp.full((B, k), -1, jnp.int32), m[:, : T - k]], axis=1)
        m = jnp.maximum(m, shifted)
        k *= 2
    prev_excl = jnp.concatenate(
        [jnp.full((B, 1), -1, jnp.int32), m[:, : T - 1]], axis=1)
    gap_ref[...] = jnp.where((valid == 1) & (prev_excl >= 0),
                             pos - prev_excl, 0)

    n = table_ref.shape[0]
    g = lax.broadcasted_iota(jnp.int32, (n, 1), 0).astype(jnp.float32)
    h = g * w1_ref[...] + b1_ref[...][None, :]
    h = 0.5 * h * (1.0 + lax.erf(h * (2.0 ** -0.5)))
    table_ref[...] = (
        jnp.dot(h, w2_ref[...], preferred_element_type=jnp.float32)
        + b2_ref[...][None, :])


def _prep(mask_i32, W1, b1, W2, b2, T):
    B = mask_i32.shape[0]
    return pl.pallas_call(
        _prep_body,
        out_shape=(
            jax.ShapeDtypeStruct((B, T), jnp.int32),
            jax.ShapeDtypeStruct((B, T), jnp.int32),
            jax.ShapeDtypeStruct((T, HALF), jnp.float32),
        ),
    )(mask_i32, W1, b1, W2, b2)


_NUM_SC_CORES = 2
_NUM_SUBCORES = 16
_NW = _NUM_SC_CORES * _NUM_SUBCORES
_CHUNK = 64


def _gather(emb, table, obs_idx, gap_idx, n_rows):
    rows_per_w = n_rows // _NW
    n_chunks = rows_per_w // _CHUNK
    mesh = plsc.VectorSubcoreMesh(core_axis_name="c", subcore_axis_name="s")

    @functools.partial(
        pl.kernel,
        mesh=mesh,
        out_type=jax.ShapeDtypeStruct((n_rows, D_MODEL), jnp.float32),
        scratch_types=[
            pltpu.VMEM((_CHUNK,), jnp.int32),
            pltpu.VMEM((_CHUNK,), jnp.int32),
            pltpu.VMEM((_CHUNK, HALF), jnp.float32),
            pltpu.VMEM((_CHUNK, HALF), jnp.float32),
            pltpu.SemaphoreType.DMA,
            pltpu.SemaphoreType.DMA,
        ],
    )
    def k(emb_hbm, table_hbm, obs_hbm, gap_hbm, out_hbm,
          idx1_v, idx2_v, buf1_v, buf2_v, sem1, sem2):
        wid = lax.axis_index("s") * _NUM_SC_CORES + lax.axis_index("c")
        wbase = wid * rows_per_w

        def body(ci, carry):
            base = pl.multiple_of(wbase + ci * _CHUNK, _CHUNK)
            pltpu.sync_copy(obs_hbm.at[pl.ds(base, _CHUNK)], idx1_v)
            pltpu.sync_copy(gap_hbm.at[pl.ds(base, _CHUNK)], idx2_v)
            cp1 = pltpu.async_copy(emb_hbm.at[idx1_v], buf1_v, sem1)
            cp2 = pltpu.async_copy(table_hbm.at[idx2_v], buf2_v, sem2)
            cp1.wait()
            cp2.wait()
            pltpu.sync_copy(buf1_v, out_hbm.at[pl.ds(base, _CHUNK),
                                               pl.ds(0, HALF)])
            pltpu.sync_copy(buf2_v, out_hbm.at[pl.ds(base, _CHUNK),
                                               pl.ds(HALF, HALF)])
            return carry

        lax.fori_loop(0, n_chunks, body, 0)

    return k(emb, table, obs_idx, gap_idx)


def kernel(x, padding_mask, emb, W1, b1, W2, b2):
    B, T, D = x.shape
    mask_i32 = padding_mask.astype(jnp.int32)
    obs_idx, gap_idx, table = _prep(mask_i32, W1, b1, W2, b2, T)
    out = _gather(emb, table, obs_idx.reshape(-1), gap_idx.reshape(-1), B * T)
    return out.reshape(B, T, D)
